# baseline (speedup 1.0000x reference)
import functools

import jax
import jax.numpy as jnp
from jax import lax
from jax.experimental import pallas as pl
from jax.experimental.pallas import tpu as pltpu
from jax.experimental.pallas import tpu_sc as plsc

V, D, B, T = 8192, 1024, 8, 3072
H, W = 32, 32

BSC = 8
BTC = B - BSC

_info = plsc.get_sparse_core_info()
NC, NS, L = _info.num_cores, _info.num_subcores, _info.num_lanes
NW = NC * NS
BLK = W
NTB = T // BLK
TB_PER_W = NTB // NW
DV = D // L
UNROLL = 8
HPB = 2
HROWS = BLK // HPB
HUNITS = TB_PER_W * BSC * HPB if BSC else 0
NBUF = 4
AHEAD = NBUF - 2


def _sc_body(idx_hbm, tok_hbm, row_hbm, col_hbm, chan_hbm, out_hbm,
             idx_v, pos_v, *rest):
    tok_bufs = rest[:NBUF]
    gsems = rest[NBUF:2 * NBUF]
    ssems = rest[2 * NBUF:3 * NBUF]
    isem = rest[3 * NBUF]
    row_v, chan_v = rest[3 * NBUF + 1], rest[3 * NBUF + 2]
    wid = lax.axis_index("s") * NC + lax.axis_index("c")

    idx_cps = []
    for k in range(TB_PER_W):
        for b in range(BSC):
            src = (wid * TB_PER_W + k) * BLK
            idx_cps.append(pltpu.async_copy(
                idx_hbm.at[b, pl.ds(src, BLK)],
                idx_v.at[pl.ds((k * BSC + b) * BLK, BLK)], isem))
    for cp in idx_cps:
        cp.wait()

    def gather(u):
        return pltpu.async_copy(
            tok_hbm.at[idx_v.at[pl.ds(u * HROWS, HROWS)]],
            tok_bufs[u % NBUF], gsems[u % NBUF])

    def build_posblk(k):
        tpos = (wid * TB_PER_W + k) * BLK
        r = (tpos % (H * W)) // W
        ch = tpos // (H * W)
        pltpu.sync_copy(row_hbm.at[r], row_v)
        pltpu.sync_copy(chan_hbm.at[ch], chan_v)
        pltpu.sync_copy(col_hbm, pos_v)

        def rc_body(i, _):
            sl = pl.ds(i * L, L)
            rc16 = row_v[sl] + chan_v[sl]

            @plsc.parallel_loop(0, BLK, unroll=8)
            def rc_j(j):
                plsc.addupdate(pos_v.at[j, sl], rc16)
            return 0
        lax.fori_loop(0, DV, rc_body, 0)

    def add_pos(buf, h):
        def add_j(j, _):
            @plsc.parallel_loop(0, DV, unroll=UNROLL)
            def add_i(i):
                sl = pl.ds(i * L, L)
                plsc.addupdate(buf.at[j, sl], pos_v[h * HROWS + j, sl])
            return 0
        lax.fori_loop(0, HROWS, add_j, 0)

    gather_cp = {}
    scatter_cp = {}
    for u in range(min(AHEAD, HUNITS)):
        gather_cp[u] = gather(u)
    for u in range(HUNITS):
        k, bh = divmod(u, BSC * HPB)
        b, h = divmod(bh, HPB)
        if bh == 0:
            build_posblk(k)
        gather_cp[u].wait()
        nxt = u + AHEAD
        if nxt < HUNITS:
            if nxt - NBUF >= 0:
                scatter_cp[nxt - NBUF].wait()
            gather_cp[nxt] = gather(nxt)
        add_pos(tok_bufs[u % NBUF], h)
        dst = b * T + (wid * TB_PER_W + k) * BLK + h * HROWS
        scatter_cp[u] = pltpu.async_copy(
            tok_bufs[u % NBUF], out_hbm.at[pl.ds(dst, HROWS)], ssems[u % NBUF])
    for u in range(max(0, HUNITS - NBUF), HUNITS):
        scatter_cp[u].wait()


def _run_sc(idx2d, tok_table, row_table, col_table, chan_table):
    mesh = plsc.VectorSubcoreMesh(core_axis_name="c", subcore_axis_name="s")
    k = functools.partial(
        pl.kernel, mesh=mesh,
        compiler_params=pltpu.CompilerParams(use_tc_tiling_on_sc=False),
        out_type=jax.ShapeDtypeStruct((BSC * T, D), jnp.float32),
        scratch_types=(
            [pltpu.VMEM((max(TB_PER_W * BSC, 1) * BLK,), jnp.int32),
             pltpu.VMEM((BLK, D), jnp.float32)]
            + [pltpu.VMEM((HROWS, D), jnp.float32)] * NBUF
            + [pltpu.SemaphoreType.DMA] * (2 * NBUF + 1)
            + [pltpu.VMEM((D,), jnp.float32),
               pltpu.VMEM((D,), jnp.float32)]
        ),
    )(_sc_body)
    return k(idx2d, tok_table, row_table, col_table, chan_table)


@jax.jit
def _run(idx2d, tok_table, row_table, col_table, chan_table):
    return _run_sc(idx2d, tok_table, row_table, col_table, chan_table)


def kernel(idx, tok_table, row_table, col_table, chan_table):
    out = _run(idx.astype(jnp.int32), tok_table, row_table, col_table,
               chan_table)
    return out.reshape(B, T, D)


# speedup vs baseline: 1.0003x; 1.0003x over previous
import functools

import jax
import jax.numpy as jnp
from jax import lax
from jax.experimental import pallas as pl
from jax.experimental.pallas import tpu as pltpu
from jax.experimental.pallas import tpu_sc as plsc

V, D, B, T = 8192, 1024, 8, 3072
H, W = 32, 32

BSC = 8
BTC = B - BSC

_info = plsc.get_sparse_core_info()
NC, NS, L = _info.num_cores, _info.num_subcores, _info.num_lanes
NW = NC * NS
BLK = W
NTB = T // BLK
TB_PER_W = NTB // NW
DV = D // L
UNROLL = 8
HPB = 2
HROWS = BLK // HPB
HUNITS = TB_PER_W * BSC * HPB if BSC else 0
NBUF = 4
AHEAD = NBUF - 2


def _sc_body(idx_hbm, tok_hbm, row_hbm, col_hbm, chan_hbm, out_hbm,
             idx_v, pos_v, *rest):
    tok_bufs = rest[:NBUF]
    gsems = rest[NBUF:2 * NBUF]
    ssems = rest[2 * NBUF:3 * NBUF]
    isem = rest[3 * NBUF]
    row_v, chan_v = rest[3 * NBUF + 1], rest[3 * NBUF + 2]
    wid = lax.axis_index("s") * NC + lax.axis_index("c")

    idx_cps = []
    for k in range(TB_PER_W):
        for b in range(BSC):
            src = (wid * TB_PER_W + k) * BLK
            idx_cps.append(pltpu.async_copy(
                idx_hbm.at[b, pl.ds(src, BLK)],
                idx_v.at[pl.ds((k * BSC + b) * BLK, BLK)], isem))
    for cp in idx_cps:
        cp.wait()

    def gather(u):
        return pltpu.async_copy(
            tok_hbm.at[idx_v.at[pl.ds(u * HROWS, HROWS)]],
            tok_bufs[u % NBUF], gsems[u % NBUF])

    def build_posblk(k):
        tpos = (wid * TB_PER_W + k) * BLK
        r = (tpos % (H * W)) // W
        ch = tpos // (H * W)
        pltpu.sync_copy(row_hbm.at[r], row_v)
        pltpu.sync_copy(chan_hbm.at[ch], chan_v)
        pltpu.sync_copy(col_hbm, pos_v)

        def rc_body(i, _):
            sl = pl.ds(i * L, L)
            rc16 = row_v[sl] + chan_v[sl]

            @plsc.parallel_loop(0, BLK, unroll=8)
            def rc_j(j):
                plsc.addupdate(pos_v.at[j, sl], rc16)
            return 0
        lax.fori_loop(0, DV, rc_body, 0)

    def add_pos(buf, h):
        def add_j(j, _):
            @plsc.parallel_loop(0, DV, unroll=UNROLL)
            def add_i(i):
                sl = pl.ds(i * L, L)
                plsc.addupdate(buf.at[j, sl], pos_v[h * HROWS + j, sl])
            return 0
        lax.fori_loop(0, HROWS, add_j, 0)

    gather_cp = {}
    scatter_cp = {}
    for u in range(min(AHEAD, HUNITS)):
        gather_cp[u] = gather(u)
    for u in range(HUNITS):
        k, bh = divmod(u, BSC * HPB)
        b, h = divmod(bh, HPB)
        if bh == 0:
            build_posblk(k)
        gather_cp[u].wait()
        nxt = u + AHEAD
        if nxt < HUNITS:
            if nxt - NBUF >= 0:
                scatter_cp[nxt - NBUF].wait()
            gather_cp[nxt] = gather(nxt)
        add_pos(tok_bufs[u % NBUF], h)
        dst = (wid * TB_PER_W + k) * BLK + h * HROWS
        scatter_cp[u] = pltpu.async_copy(
            tok_bufs[u % NBUF], out_hbm.at[b, pl.ds(dst, HROWS)],
            ssems[u % NBUF])
    for u in range(max(0, HUNITS - NBUF), HUNITS):
        scatter_cp[u].wait()


def _run_sc(idx2d, tok_table, row_table, col_table, chan_table):
    mesh = plsc.VectorSubcoreMesh(core_axis_name="c", subcore_axis_name="s")
    k = functools.partial(
        pl.kernel, mesh=mesh,
        compiler_params=pltpu.CompilerParams(use_tc_tiling_on_sc=False),
        out_type=jax.ShapeDtypeStruct((B, T, D), jnp.float32),
        scratch_types=(
            [pltpu.VMEM((max(TB_PER_W * BSC, 1) * BLK,), jnp.int32),
             pltpu.VMEM((BLK, D), jnp.float32)]
            + [pltpu.VMEM((HROWS, D), jnp.float32)] * NBUF
            + [pltpu.SemaphoreType.DMA] * (2 * NBUF + 1)
            + [pltpu.VMEM((D,), jnp.float32),
               pltpu.VMEM((D,), jnp.float32)]
        ),
    )(_sc_body)
    return k(idx2d, tok_table, row_table, col_table, chan_table)


@jax.jit
def _run(idx2d, tok_table, row_table, col_table, chan_table):
    return _run_sc(idx2d, tok_table, row_table, col_table, chan_table)


def kernel(idx, tok_table, row_table, col_table, chan_table):
    out = _run(idx.astype(jnp.int32), tok_table, row_table, col_table,
               chan_table)
    return out


# speedup vs baseline: 2.0653x; 2.0646x over previous
import functools

import jax
import jax.numpy as jnp
from jax import lax
from jax.experimental import pallas as pl
from jax.experimental.pallas import tpu as pltpu
from jax.experimental.pallas import tpu_sc as plsc

V, D, B, T = 8192, 1024, 8, 3072
H, W = 32, 32

BSC = 8
BTC = B - BSC

_info = plsc.get_sparse_core_info()
NC, NS, L = _info.num_cores, _info.num_subcores, _info.num_lanes
NW = NC * NS
BLK = W
NTB = T // BLK
TB_PER_W = NTB // NW
DV = D // L
UNROLL = 8
HPB = 2
HROWS = BLK // HPB
HUNITS = TB_PER_W * BSC * HPB if BSC else 0
NBUF = 4
AHEAD = NBUF - 2


def _sc_body(idx_hbm, tok_hbm, row_hbm, col_hbm, chan_hbm, out_hbm,
             idx_v, pos_v, *rest):
    tok_bufs = rest[:NBUF]
    gsems = rest[NBUF:2 * NBUF]
    ssems = rest[2 * NBUF:3 * NBUF]
    isem = rest[3 * NBUF]
    row_v, chan_v = rest[3 * NBUF + 1], rest[3 * NBUF + 2]
    wid = lax.axis_index("s") * NC + lax.axis_index("c")

    idx_cps = []
    for k in range(TB_PER_W):
        for b in range(BSC):
            src = (wid * TB_PER_W + k) * BLK
            idx_cps.append(pltpu.async_copy(
                idx_hbm.at[b, pl.ds(src, BLK)],
                idx_v.at[pl.ds((k * BSC + b) * BLK, BLK)], isem))
    for cp in idx_cps:
        cp.wait()

    def gather(u):
        return pltpu.async_copy(
            tok_hbm.at[idx_v.at[pl.ds(u * HROWS, HROWS)]],
            tok_bufs[u % NBUF], gsems[u % NBUF])

    def build_posblk(k):
        tpos = (wid * TB_PER_W + k) * BLK
        r = (tpos % (H * W)) // W
        ch = tpos // (H * W)
        pltpu.sync_copy(row_hbm.at[r], row_v)
        pltpu.sync_copy(chan_hbm.at[ch], chan_v)
        pltpu.sync_copy(col_hbm, pos_v)

        def rc_body(i, _):
            sl = pl.ds(i * L, L)
            rc16 = row_v[sl] + chan_v[sl]

            @plsc.parallel_loop(0, BLK, unroll=8)
            def rc_j(j):
                plsc.addupdate(pos_v.at[j, sl], rc16)
            return 0
        lax.fori_loop(0, DV, rc_body, 0)

    def add_pos(buf, h):
        def add_j(j, _):
            @plsc.parallel_loop(0, DV, unroll=UNROLL)
            def add_i(i):
                sl = pl.ds(i * L, L)
                plsc.addupdate(buf.at[j, sl], pos_v[h * HROWS + j, sl])
            return 0
        lax.fori_loop(0, HROWS, add_j, 0)

    gather_cp = {}
    scatter_cp = {}
    for u in range(min(AHEAD, HUNITS)):
        gather_cp[u] = gather(u)
    for u in range(HUNITS):
        k, bh = divmod(u, BSC * HPB)
        b, h = divmod(bh, HPB)
        if bh == 0:
            build_posblk(k)
        gather_cp[u].wait()
        nxt = u + AHEAD
        if nxt < HUNITS:
            if nxt - NBUF >= 0:
                scatter_cp[nxt - NBUF].wait()
            gather_cp[nxt] = gather(nxt)
        add_pos(tok_bufs[u % NBUF], h)
        dst = (wid * TB_PER_W + k) * BLK + h * HROWS
        scatter_cp[u] = pltpu.async_copy(
            tok_bufs[u % NBUF], out_hbm.at[b, pl.ds(dst, HROWS)],
            ssems[u % NBUF])
    for u in range(max(0, HUNITS - NBUF), HUNITS):
        scatter_cp[u].wait()


def _run_sc(idx2d, tok_table, row_table, col_table, chan_table):
    mesh = plsc.VectorSubcoreMesh(core_axis_name="c", subcore_axis_name="s")
    k = functools.partial(
        pl.kernel, mesh=mesh,
        out_type=jax.ShapeDtypeStruct((B, T, D), jnp.float32),
        scratch_types=(
            [pltpu.VMEM((max(TB_PER_W * BSC, 1) * BLK,), jnp.int32),
             pltpu.VMEM((BLK, D), jnp.float32)]
            + [pltpu.VMEM((HROWS, D), jnp.float32)] * NBUF
            + [pltpu.SemaphoreType.DMA] * (2 * NBUF + 1)
            + [pltpu.VMEM((D,), jnp.float32),
               pltpu.VMEM((D,), jnp.float32)]
        ),
    )(_sc_body)
    return k(idx2d, tok_table, row_table, col_table, chan_table)


@jax.jit
def _run(idx2d, tok_table, row_table, col_table, chan_table):
    return _run_sc(idx2d, tok_table, row_table, col_table, chan_table)


def kernel(idx, tok_table, row_table, col_table, chan_table):
    out = _run(idx.astype(jnp.int32), tok_table, row_table, col_table,
               chan_table)
    return out
